# Initial kernel scaffold; baseline (speedup 1.0000x reference)
import functools

import jax
import jax.numpy as jnp
from jax import lax
from jax.experimental import pallas as pl
from jax.experimental.pallas import tpu as pltpu
from jax.experimental.pallas import tpu_sc as plsc

N = 10000
NPAD = 10240
D_IN = 128
H1 = 64
H2 = 32
OUT = 2

NC = 2
NS = 16
NW = NC * NS
CHUNK = 128
ROWS_PER_SUB = NPAD // NS


def _worker_id():
    return lax.axis_index("s") * NC + lax.axis_index("c")


def _deg_body(chunks_per_worker, dst_hbm, out_hbm, idx_v, ones_v, zer_v, acc_sh):
    cid = lax.axis_index("c")
    sid = lax.axis_index("s")
    wid = _worker_id()
    for i in range(CHUNK // 16):
        ones_v[pl.ds(i * 16, 16)] = jnp.ones((16,), jnp.float32)
    for i in range(ROWS_PER_SUB // 16):
        zer_v[pl.ds(i * 16, 16)] = jnp.zeros((16,), jnp.float32)
    pltpu.sync_copy(zer_v, acc_sh.at[pl.ds(sid * ROWS_PER_SUB, ROWS_PER_SUB)])
    plsc.subcore_barrier()

    def chunk(k, carry):
        base = (wid * chunks_per_worker + k) * CHUNK
        pltpu.sync_copy(dst_hbm.at[pl.ds(base, CHUNK)], idx_v)
        pltpu.sync_copy(ones_v, acc_sh.at[idx_v], add=True)
        return carry

    lax.fori_loop(0, chunks_per_worker, chunk, 0)
    plsc.subcore_barrier()
    sl = pl.ds(sid * ROWS_PER_SUB, ROWS_PER_SUB)
    pltpu.sync_copy(acc_sh.at[sl], out_hbm.at[cid, sl])


def _make_deg_kernel(epad):
    chunks_per_worker = epad // (NW * CHUNK)
    mesh = plsc.VectorSubcoreMesh(core_axis_name="c", subcore_axis_name="s")
    return pl.kernel(
        functools.partial(_deg_body, chunks_per_worker),
        out_type=jax.ShapeDtypeStruct((NC, NPAD), jnp.float32),
        mesh=mesh,
        scratch_types=[
            pltpu.VMEM((CHUNK,), jnp.int32),
            pltpu.VMEM((CHUNK,), jnp.float32),
            pltpu.VMEM((ROWS_PER_SUB,), jnp.float32),
            pltpu.VMEM_SHARED((NPAD,), jnp.float32),
        ],
    )


def _agg_body(chunks_per_worker, hp_hbm, src_hbm, dst_hbm, out_hbm,
              sidx_v, didx_v, rows_v, acc_sh):
    cid = lax.axis_index("c")
    sid = lax.axis_index("s")
    wid = _worker_id()
    sl = pl.ds(sid * ROWS_PER_SUB, ROWS_PER_SUB)
    pltpu.sync_copy(hp_hbm.at[sl], acc_sh.at[sl])
    plsc.subcore_barrier()

    def chunk(k, carry):
        base = (wid * chunks_per_worker + k) * CHUNK
        pltpu.sync_copy(src_hbm.at[pl.ds(base, CHUNK)], sidx_v)
        pltpu.sync_copy(dst_hbm.at[pl.ds(base, CHUNK)], didx_v)
        pltpu.sync_copy(hp_hbm.at[sidx_v], rows_v)
        pltpu.sync_copy(rows_v, acc_sh.at[didx_v], add=True)
        return carry

    lax.fori_loop(0, chunks_per_worker, chunk, 0)
    plsc.subcore_barrier()
    pltpu.sync_copy(acc_sh.at[sl], out_hbm.at[cid, sl])


def _make_agg_kernel(epad, h):
    chunks_per_worker = epad // (NW * CHUNK)
    mesh = plsc.VectorSubcoreMesh(core_axis_name="c", subcore_axis_name="s")
    return pl.kernel(
        functools.partial(_agg_body, chunks_per_worker),
        out_type=jax.ShapeDtypeStruct((NC, NPAD, h), jnp.float32),
        mesh=mesh,
        scratch_types=[
            pltpu.VMEM((CHUNK,), jnp.int32),
            pltpu.VMEM((CHUNK,), jnp.int32),
            pltpu.VMEM((CHUNK, h), jnp.float32),
            pltpu.VMEM_SHARED((NPAD, h), jnp.float32),
        ],
    )


def _tc1_body(x_ref, w1_ref, degp_ref, hp_ref, dinv_ref):
    deg = degp_ref[:, 0:1] + degp_ref[:, 1:2] + 1.0
    dinv = lax.rsqrt(deg)
    dinv_ref[...] = dinv
    hp_ref[...] = jnp.dot(x_ref[...], w1_ref[...],
                          preferred_element_type=jnp.float32) * dinv


def _row_mask(t):
    rid = lax.broadcasted_iota(jnp.int32, t.shape, 0)
    return rid < N


def _bn_relu(t, g, b):
    msk = _row_mask(t)
    tm = jnp.where(msk, t, 0.0)
    m = jnp.sum(tm, axis=0, keepdims=True) / N
    v = jnp.sum(jnp.where(msk, (t - m) ** 2, 0.0), axis=0, keepdims=True) / N
    return jnp.maximum((t - m) * lax.rsqrt(v + 1e-5) * g + b, 0.0)


def _tc2_body(p_ref, hp_ref, dinv_ref, b1_ref, g1_ref, be1_ref, w2_ref,
              hp2_ref):
    t = p_ref[0] + p_ref[1] - hp_ref[...]
    agg = t * dinv_ref[...] + b1_ref[...]
    a1 = _bn_relu(agg, g1_ref[...], be1_ref[...])
    hp2_ref[...] = jnp.dot(a1, w2_ref[...],
                           preferred_element_type=jnp.float32) * dinv_ref[...]


def _tc3_body(p_ref, hp_ref, dinv_ref, b2_ref, g2_ref, be2_ref, wfc_ref,
              bfc_ref, out_ref):
    t = p_ref[0] + p_ref[1] - hp_ref[...]
    agg = t * dinv_ref[...] + b2_ref[...]
    a2 = _bn_relu(agg, g2_ref[...], be2_ref[...])
    out_ref[...] = jnp.dot(a2, wfc_ref[...],
                           preferred_element_type=jnp.float32) + bfc_ref[...]


def kernel(x, edge_index, W1, b1, g1, be1, W2, b2, g2, be2, Wfc, bfc):
    e = edge_index.shape[1]
    epad = ((e + NW * CHUNK - 1) // (NW * CHUNK)) * (NW * CHUNK)
    pad = epad - e
    src = edge_index[0]
    dst = edge_index[1]
    if pad:
        fill = jnp.arange(pad, dtype=jnp.int32)
        src = jnp.concatenate([src, fill % N])
        dst = jnp.concatenate([dst, N + fill % (NPAD - N)])
    x_pad = jnp.pad(x, ((0, NPAD - N), (0, 0)))

    degp = _make_deg_kernel(epad)(dst)

    hp1, dinv = pl.pallas_call(
        _tc1_body,
        out_shape=[
            jax.ShapeDtypeStruct((NPAD, H1), jnp.float32),
            jax.ShapeDtypeStruct((NPAD, 1), jnp.float32),
        ],
    )(x_pad, W1, degp.T)

    p1 = _make_agg_kernel(epad, H1)(hp1, src, dst)

    hp2 = pl.pallas_call(
        _tc2_body,
        out_shape=jax.ShapeDtypeStruct((NPAD, H2), jnp.float32),
    )(p1, hp1, dinv, b1, g1, be1, W2)

    p2 = _make_agg_kernel(epad, H2)(hp2, src, dst)

    logits = pl.pallas_call(
        _tc3_body,
        out_shape=jax.ShapeDtypeStruct((NPAD, OUT), jnp.float32),
    )(p2, hp2, dinv, b2, g2, be2, Wfc, bfc)

    return logits[:N]


# speedup vs baseline: 19.8054x; 19.8054x over previous
import functools

import jax
import jax.numpy as jnp
from jax import lax
from jax.experimental import pallas as pl
from jax.experimental.pallas import tpu as pltpu
from jax.experimental.pallas import tpu_sc as plsc

N = 10000
NPAD = 10240
D_IN = 128
H1 = 64
H2 = 32
OUT = 2

NC = 2
NS = 16
NW = NC * NS
CHUNK = 128
ROWS_PER_SUB = NPAD // NS


def _worker_id():
    return lax.axis_index("s") * NC + lax.axis_index("c")


def _deg_body(chunks_per_worker, dst_hbm, out_hbm, idx_v, ones_v, zer_v, acc_sh):
    cid = lax.axis_index("c")
    sid = lax.axis_index("s")
    wid = _worker_id()
    for i in range(CHUNK // 16):
        ones_v[pl.ds(i * 16, 16)] = jnp.ones((16,), jnp.float32)
    for i in range(ROWS_PER_SUB // 16):
        zer_v[pl.ds(i * 16, 16)] = jnp.zeros((16,), jnp.float32)
    pltpu.sync_copy(zer_v, acc_sh.at[pl.ds(sid * ROWS_PER_SUB, ROWS_PER_SUB)])
    plsc.subcore_barrier()

    def chunk(k, carry):
        base = (wid * chunks_per_worker + k) * CHUNK
        pltpu.sync_copy(dst_hbm.at[pl.ds(base, CHUNK)], idx_v)
        pltpu.sync_copy(ones_v, acc_sh.at[idx_v], add=True)
        return carry

    lax.fori_loop(0, chunks_per_worker, chunk, 0)
    plsc.subcore_barrier()
    sl = pl.ds(sid * ROWS_PER_SUB, ROWS_PER_SUB)
    pltpu.sync_copy(acc_sh.at[sl], out_hbm.at[cid, sl])


def _make_deg_kernel(epad):
    chunks_per_worker = epad // (NW * CHUNK)
    mesh = plsc.VectorSubcoreMesh(core_axis_name="c", subcore_axis_name="s")
    return pl.kernel(
        functools.partial(_deg_body, chunks_per_worker),
        out_type=jax.ShapeDtypeStruct((NC, NPAD), jnp.float32),
        mesh=mesh,
        scratch_types=[
            pltpu.VMEM((CHUNK,), jnp.int32),
            pltpu.VMEM((CHUNK,), jnp.float32),
            pltpu.VMEM((ROWS_PER_SUB,), jnp.float32),
            pltpu.VMEM_SHARED((NPAD,), jnp.float32),
        ],
    )


def _agg_body(chunks_per_worker, hp_hbm, src_hbm, dst_hbm, out_hbm,
              sidx_v, didx_v, rows_v, acc_sh):
    cid = lax.axis_index("c")
    sid = lax.axis_index("s")
    wid = _worker_id()
    sl = pl.ds(sid * ROWS_PER_SUB, ROWS_PER_SUB)
    pltpu.sync_copy(hp_hbm.at[sl], acc_sh.at[sl])
    plsc.subcore_barrier()

    def chunk(k, carry):
        base = (wid * chunks_per_worker + k) * CHUNK
        pltpu.sync_copy(src_hbm.at[pl.ds(base, CHUNK)], sidx_v)
        pltpu.sync_copy(dst_hbm.at[pl.ds(base, CHUNK)], didx_v)
        pltpu.sync_copy(hp_hbm.at[sidx_v], rows_v)
        pltpu.sync_copy(rows_v, acc_sh.at[didx_v], add=True)
        return carry

    lax.fori_loop(0, chunks_per_worker, chunk, 0)
    plsc.subcore_barrier()
    pltpu.sync_copy(acc_sh.at[sl], out_hbm.at[cid, sl])


def _make_agg_kernel(epad, h):
    chunks_per_worker = epad // (NW * CHUNK)
    mesh = plsc.VectorSubcoreMesh(core_axis_name="c", subcore_axis_name="s")
    return pl.kernel(
        functools.partial(_agg_body, chunks_per_worker),
        out_type=jax.ShapeDtypeStruct((NC, NPAD, h), jnp.float32),
        mesh=mesh,
        scratch_types=[
            pltpu.VMEM((CHUNK,), jnp.int32),
            pltpu.VMEM((CHUNK,), jnp.int32),
            pltpu.VMEM((CHUNK, h), jnp.float32),
            pltpu.VMEM_SHARED((NPAD, h), jnp.float32),
        ],
        compiler_params=pltpu.CompilerParams(use_tc_tiling_on_sc=False),
    )


def _tc1_body(x_ref, w1_ref, degp_ref, hp_ref, dinv_ref):
    deg = degp_ref[:, 0:1] + degp_ref[:, 1:2] + 1.0
    dinv = lax.rsqrt(deg)
    dinv_ref[...] = dinv
    hp_ref[...] = jnp.dot(x_ref[...], w1_ref[...],
                          preferred_element_type=jnp.float32) * dinv


def _row_mask(t):
    rid = lax.broadcasted_iota(jnp.int32, t.shape, 0)
    return rid < N


def _bn_relu(t, g, b):
    msk = _row_mask(t)
    tm = jnp.where(msk, t, 0.0)
    m = jnp.sum(tm, axis=0, keepdims=True) / N
    v = jnp.sum(jnp.where(msk, (t - m) ** 2, 0.0), axis=0, keepdims=True) / N
    return jnp.maximum((t - m) * lax.rsqrt(v + 1e-5) * g + b, 0.0)


def _tc2_body(p_ref, hp_ref, dinv_ref, b1_ref, g1_ref, be1_ref, w2_ref,
              hp2_ref):
    t = p_ref[0] + p_ref[1] - hp_ref[...]
    agg = t * dinv_ref[...] + b1_ref[...]
    a1 = _bn_relu(agg, g1_ref[...], be1_ref[...])
    hp2_ref[...] = jnp.dot(a1, w2_ref[...],
                           preferred_element_type=jnp.float32) * dinv_ref[...]


def _tc3_body(p_ref, hp_ref, dinv_ref, b2_ref, g2_ref, be2_ref, wfc_ref,
              bfc_ref, out_ref):
    t = p_ref[0] + p_ref[1] - hp_ref[...]
    agg = t * dinv_ref[...] + b2_ref[...]
    a2 = _bn_relu(agg, g2_ref[...], be2_ref[...])
    out_ref[...] = jnp.dot(a2, wfc_ref[...],
                           preferred_element_type=jnp.float32) + bfc_ref[...]


def kernel(x, edge_index, W1, b1, g1, be1, W2, b2, g2, be2, Wfc, bfc):
    e = edge_index.shape[1]
    epad = ((e + NW * CHUNK - 1) // (NW * CHUNK)) * (NW * CHUNK)
    pad = epad - e
    src = edge_index[0]
    dst = edge_index[1]
    if pad:
        fill = jnp.arange(pad, dtype=jnp.int32)
        src = jnp.concatenate([src, fill % N])
        dst = jnp.concatenate([dst, N + fill % (NPAD - N)])
    x_pad = jnp.pad(x, ((0, NPAD - N), (0, 0)))

    degp = _make_deg_kernel(epad)(dst)

    hp1, dinv = pl.pallas_call(
        _tc1_body,
        out_shape=[
            jax.ShapeDtypeStruct((NPAD, H1), jnp.float32),
            jax.ShapeDtypeStruct((NPAD, 1), jnp.float32),
        ],
    )(x_pad, W1, degp.T)

    p1 = _make_agg_kernel(epad, H1)(hp1, src, dst)

    hp2 = pl.pallas_call(
        _tc2_body,
        out_shape=jax.ShapeDtypeStruct((NPAD, H2), jnp.float32),
    )(p1, hp1, dinv, b1, g1, be1, W2)

    p2 = _make_agg_kernel(epad, H2)(hp2, src, dst)

    logits = pl.pallas_call(
        _tc3_body,
        out_shape=jax.ShapeDtypeStruct((NPAD, OUT), jnp.float32),
    )(p2, hp2, dinv, b2, g2, be2, Wfc, bfc)

    return logits[:N]


# speedup vs baseline: 41.6965x; 2.1053x over previous
import functools

import jax
import jax.numpy as jnp
from jax import lax
from jax.experimental import pallas as pl
from jax.experimental.pallas import tpu as pltpu
from jax.experimental.pallas import tpu_sc as plsc

N = 10000
NPAD = 10240
D_IN = 128
H1 = 64
H2 = 32
OUT = 2

NC = 2
NS = 16
NW = NC * NS
CHUNK = 128
U = 8
ROWS_PER_SUB = NPAD // NS

_SC_PARAMS = pltpu.CompilerParams(use_tc_tiling_on_sc=False)


def _worker_id():
    return lax.axis_index("s") * NC + lax.axis_index("c")


def _deg_body(chunks_per_worker, eidx_hbm, out_hbm,
              didx_v, ones_v, zer_v, isem, ssem, acc_sh):
    cid = lax.axis_index("c")
    sid = lax.axis_index("s")
    wid = _worker_id()
    for i in range(CHUNK // 16):
        ones_v[pl.ds(i * 16, 16)] = jnp.ones((16,), jnp.float32)
    for i in range(ROWS_PER_SUB // 16):
        zer_v[pl.ds(i * 16, 16)] = jnp.zeros((16,), jnp.float32)
    pltpu.sync_copy(zer_v, acc_sh.at[pl.ds(sid * ROWS_PER_SUB, ROWS_PER_SUB)])
    plsc.subcore_barrier()

    def group(g, carry):
        cbase = wid * chunks_per_worker + g * U
        loads = [
            pltpu.async_copy(eidx_hbm.at[cbase + b, 1], didx_v.at[b],
                             isem.at[b])
            for b in range(U)
        ]
        scats = []
        for b in range(U):
            loads[b].wait()
            scats.append(pltpu.async_copy(
                ones_v, acc_sh.at[didx_v.at[b]], ssem.at[b], add=True))
        for b in range(U):
            scats[b].wait()
        return carry

    lax.fori_loop(0, chunks_per_worker // U, group, 0)
    plsc.subcore_barrier()
    sl = pl.ds(sid * ROWS_PER_SUB, ROWS_PER_SUB)
    pltpu.sync_copy(acc_sh.at[sl], out_hbm.at[cid, sl])


def _make_deg_kernel(n_chunks):
    chunks_per_worker = n_chunks // NW
    mesh = plsc.VectorSubcoreMesh(core_axis_name="c", subcore_axis_name="s")
    return pl.kernel(
        functools.partial(_deg_body, chunks_per_worker),
        out_type=jax.ShapeDtypeStruct((NC, NPAD), jnp.float32),
        mesh=mesh,
        scratch_types=[
            pltpu.VMEM((U, CHUNK), jnp.int32),
            pltpu.VMEM((CHUNK,), jnp.float32),
            pltpu.VMEM((ROWS_PER_SUB,), jnp.float32),
            pltpu.SemaphoreType.DMA((U,)),
            pltpu.SemaphoreType.DMA((U,)),
            pltpu.VMEM_SHARED((NPAD,), jnp.float32),
        ],
        compiler_params=_SC_PARAMS,
    )


def _agg_body(chunks_per_worker, hp_hbm, eidx_hbm, out_hbm,
              eidx_v, rows_v, isem, gsem, ssem, acc_sh):
    cid = lax.axis_index("c")
    sid = lax.axis_index("s")
    wid = _worker_id()
    sl = pl.ds(sid * ROWS_PER_SUB, ROWS_PER_SUB)
    pltpu.sync_copy(hp_hbm.at[sl], acc_sh.at[sl])
    plsc.subcore_barrier()

    def group(g, carry):
        cbase = wid * chunks_per_worker + g * U
        loads = [
            pltpu.async_copy(eidx_hbm.at[cbase + b], eidx_v.at[b], isem.at[b])
            for b in range(U)
        ]
        gats = []
        for b in range(U):
            loads[b].wait()
            gats.append(pltpu.async_copy(
                hp_hbm.at[eidx_v.at[b, 0]], rows_v.at[b], gsem.at[b]))
        scats = []
        for b in range(U):
            gats[b].wait()
            scats.append(pltpu.async_copy(
                rows_v.at[b], acc_sh.at[eidx_v.at[b, 1]], ssem.at[b],
                add=True))
        for b in range(U):
            scats[b].wait()
        return carry

    lax.fori_loop(0, chunks_per_worker // U, group, 0)
    plsc.subcore_barrier()
    pltpu.sync_copy(acc_sh.at[sl], out_hbm.at[cid, sl])


def _make_agg_kernel(n_chunks, h):
    chunks_per_worker = n_chunks // NW
    mesh = plsc.VectorSubcoreMesh(core_axis_name="c", subcore_axis_name="s")
    return pl.kernel(
        functools.partial(_agg_body, chunks_per_worker),
        out_type=jax.ShapeDtypeStruct((NC, NPAD, h), jnp.float32),
        mesh=mesh,
        scratch_types=[
            pltpu.VMEM((U, 2, CHUNK), jnp.int32),
            pltpu.VMEM((U, CHUNK, h), jnp.float32),
            pltpu.SemaphoreType.DMA((U,)),
            pltpu.SemaphoreType.DMA((U,)),
            pltpu.SemaphoreType.DMA((U,)),
            pltpu.VMEM_SHARED((NPAD, h), jnp.float32),
        ],
        compiler_params=_SC_PARAMS,
    )


def _tc1_body(x_ref, w1_ref, degp_ref, hp_ref, dinv_ref):
    deg = degp_ref[:, 0:1] + degp_ref[:, 1:2] + 1.0
    dinv = lax.rsqrt(deg)
    dinv_ref[...] = dinv
    hp_ref[...] = jnp.dot(x_ref[...], w1_ref[...],
                          preferred_element_type=jnp.float32) * dinv


def _row_mask(t):
    rid = lax.broadcasted_iota(jnp.int32, t.shape, 0)
    return rid < N


def _bn_relu(t, g, b):
    msk = _row_mask(t)
    tm = jnp.where(msk, t, 0.0)
    m = jnp.sum(tm, axis=0, keepdims=True) / N
    v = jnp.sum(jnp.where(msk, (t - m) ** 2, 0.0), axis=0, keepdims=True) / N
    return jnp.maximum((t - m) * lax.rsqrt(v + 1e-5) * g + b, 0.0)


def _tc2_body(p_ref, hp_ref, dinv_ref, b1_ref, g1_ref, be1_ref, w2_ref,
              hp2_ref):
    t = p_ref[0] + p_ref[1] - hp_ref[...]
    agg = t * dinv_ref[...] + b1_ref[...]
    a1 = _bn_relu(agg, g1_ref[...], be1_ref[...])
    hp2_ref[...] = jnp.dot(a1, w2_ref[...],
                           preferred_element_type=jnp.float32) * dinv_ref[...]


def _tc3_body(p_ref, hp_ref, dinv_ref, b2_ref, g2_ref, be2_ref, wfc_ref,
              bfc_ref, out_ref):
    t = p_ref[0] + p_ref[1] - hp_ref[...]
    agg = t * dinv_ref[...] + b2_ref[...]
    a2 = _bn_relu(agg, g2_ref[...], be2_ref[...])
    out_ref[...] = jnp.dot(a2, wfc_ref[...],
                           preferred_element_type=jnp.float32) + bfc_ref[...]


def kernel(x, edge_index, W1, b1, g1, be1, W2, b2, g2, be2, Wfc, bfc):
    e = edge_index.shape[1]
    egrp = NW * CHUNK * U
    epad = ((e + egrp - 1) // egrp) * egrp
    pad = epad - e
    src = edge_index[0]
    dst = edge_index[1]
    if pad:
        fill = jnp.arange(pad, dtype=jnp.int32)
        src = jnp.concatenate([src, fill % N])
        dst = jnp.concatenate([dst, N + fill % (NPAD - N)])
    n_chunks = epad // CHUNK
    eidx = jnp.stack([src.reshape(n_chunks, CHUNK),
                      dst.reshape(n_chunks, CHUNK)], axis=1)
    x_pad = jnp.pad(x, ((0, NPAD - N), (0, 0)))

    degp = _make_deg_kernel(n_chunks)(eidx)

    hp1, dinv = pl.pallas_call(
        _tc1_body,
        out_shape=[
            jax.ShapeDtypeStruct((NPAD, H1), jnp.float32),
            jax.ShapeDtypeStruct((NPAD, 1), jnp.float32),
        ],
    )(x_pad, W1, degp.T)

    p1 = _make_agg_kernel(n_chunks, H1)(hp1, eidx)

    hp2 = pl.pallas_call(
        _tc2_body,
        out_shape=jax.ShapeDtypeStruct((NPAD, H2), jnp.float32),
    )(p1, hp1, dinv, b1, g1, be1, W2)

    p2 = _make_agg_kernel(n_chunks, H2)(hp2, eidx)

    logits = pl.pallas_call(
        _tc3_body,
        out_shape=jax.ShapeDtypeStruct((NPAD, OUT), jnp.float32),
    )(p2, hp2, dinv, b2, g2, be2, Wfc, bfc)

    return logits[:N]


# speedup vs baseline: 44.1698x; 1.0593x over previous
import functools

import jax
import jax.numpy as jnp
from jax import lax
from jax.experimental import pallas as pl
from jax.experimental.pallas import tpu as pltpu
from jax.experimental.pallas import tpu_sc as plsc

N = 10000
NPAD = 10240
D_IN = 128
H1 = 64
H2 = 32
OUT = 2

NC = 2
NS = 16
NW = NC * NS
CHUNK = 128
U = 8
ROWS_PER_SUB = NPAD // NS

_SC_PARAMS = pltpu.CompilerParams(use_tc_tiling_on_sc=False)


def _worker_id():
    return lax.axis_index("s") * NC + lax.axis_index("c")


def _deg_body(chunks_per_worker, eidx_hbm, out_hbm,
              eidx_all, ones_v, zer_v, ssem, acc_sh):
    cid = lax.axis_index("c")
    sid = lax.axis_index("s")
    wid = _worker_id()
    for i in range(CHUNK // 16):
        ones_v[pl.ds(i * 16, 16)] = jnp.ones((16,), jnp.float32)
    for i in range(ROWS_PER_SUB // 16):
        zer_v[pl.ds(i * 16, 16)] = jnp.zeros((16,), jnp.float32)
    pltpu.sync_copy(
        eidx_hbm.at[pl.ds(wid * chunks_per_worker, chunks_per_worker)],
        eidx_all)
    pltpu.sync_copy(zer_v, acc_sh.at[pl.ds(sid * ROWS_PER_SUB, ROWS_PER_SUB)])
    plsc.subcore_barrier()

    def group(g, carry):
        scats = [
            pltpu.async_copy(
                ones_v, acc_sh.at[eidx_all.at[g * U + b, 1]], ssem.at[b],
                add=True)
            for b in range(U)
        ]
        for b in range(U):
            scats[b].wait()
        return carry

    lax.fori_loop(0, chunks_per_worker // U, group, 0)
    plsc.subcore_barrier()
    sl = pl.ds(sid * ROWS_PER_SUB, ROWS_PER_SUB)
    pltpu.sync_copy(acc_sh.at[sl], out_hbm.at[cid, sl])


def _make_deg_kernel(n_chunks):
    chunks_per_worker = n_chunks // NW
    mesh = plsc.VectorSubcoreMesh(core_axis_name="c", subcore_axis_name="s")
    return pl.kernel(
        functools.partial(_deg_body, chunks_per_worker),
        out_type=jax.ShapeDtypeStruct((NC, NPAD), jnp.float32),
        mesh=mesh,
        scratch_types=[
            pltpu.VMEM((chunks_per_worker, 2, CHUNK), jnp.int32),
            pltpu.VMEM((CHUNK,), jnp.float32),
            pltpu.VMEM((ROWS_PER_SUB,), jnp.float32),
            pltpu.SemaphoreType.DMA((U,)),
            pltpu.VMEM_SHARED((NPAD,), jnp.float32),
        ],
        compiler_params=_SC_PARAMS,
    )


def _agg_body(chunks_per_worker, hp_hbm, eidx_hbm, out_hbm,
              eidx_all, rows_v, gsem, ssem, acc_sh):
    cid = lax.axis_index("c")
    sid = lax.axis_index("s")
    wid = _worker_id()
    sl = pl.ds(sid * ROWS_PER_SUB, ROWS_PER_SUB)
    pltpu.sync_copy(
        eidx_hbm.at[pl.ds(wid * chunks_per_worker, chunks_per_worker)],
        eidx_all)
    pltpu.sync_copy(hp_hbm.at[sl], acc_sh.at[sl])
    plsc.subcore_barrier()

    def group(g, carry):
        gats = [
            pltpu.async_copy(
                hp_hbm.at[eidx_all.at[g * U + b, 0]], rows_v.at[b],
                gsem.at[b])
            for b in range(U)
        ]
        scats = []
        for b in range(U):
            gats[b].wait()
            scats.append(pltpu.async_copy(
                rows_v.at[b], acc_sh.at[eidx_all.at[g * U + b, 1]],
                ssem.at[b], add=True))
        for b in range(U):
            scats[b].wait()
        return carry

    lax.fori_loop(0, chunks_per_worker // U, group, 0)
    plsc.subcore_barrier()
    pltpu.sync_copy(acc_sh.at[sl], out_hbm.at[cid, sl])


def _make_agg_kernel(n_chunks, h):
    chunks_per_worker = n_chunks // NW
    mesh = plsc.VectorSubcoreMesh(core_axis_name="c", subcore_axis_name="s")
    return pl.kernel(
        functools.partial(_agg_body, chunks_per_worker),
        out_type=jax.ShapeDtypeStruct((NC, NPAD, h), jnp.float32),
        mesh=mesh,
        scratch_types=[
            pltpu.VMEM((chunks_per_worker, 2, CHUNK), jnp.int32),
            pltpu.VMEM((U, CHUNK, h), jnp.float32),
            pltpu.SemaphoreType.DMA((U,)),
            pltpu.SemaphoreType.DMA((U,)),
            pltpu.VMEM_SHARED((NPAD, h), jnp.float32),
        ],
        compiler_params=_SC_PARAMS,
    )


def _tc1_body(x_ref, w1_ref, degp_ref, hp_ref, dinv_ref):
    deg = degp_ref[:, 0:1] + degp_ref[:, 1:2] + 1.0
    dinv = lax.rsqrt(deg)
    dinv_ref[...] = dinv
    hp_ref[...] = jnp.dot(x_ref[...], w1_ref[...],
                          preferred_element_type=jnp.float32) * dinv


def _row_mask(t):
    rid = lax.broadcasted_iota(jnp.int32, t.shape, 0)
    return rid < N


def _bn_relu(t, g, b):
    msk = _row_mask(t)
    tm = jnp.where(msk, t, 0.0)
    m = jnp.sum(tm, axis=0, keepdims=True) / N
    v = jnp.sum(jnp.where(msk, (t - m) ** 2, 0.0), axis=0, keepdims=True) / N
    return jnp.maximum((t - m) * lax.rsqrt(v + 1e-5) * g + b, 0.0)


def _tc2_body(p_ref, hp_ref, dinv_ref, b1_ref, g1_ref, be1_ref, w2_ref,
              hp2_ref):
    t = p_ref[0] + p_ref[1] - hp_ref[...]
    agg = t * dinv_ref[...] + b1_ref[...]
    a1 = _bn_relu(agg, g1_ref[...], be1_ref[...])
    hp2_ref[...] = jnp.dot(a1, w2_ref[...],
                           preferred_element_type=jnp.float32) * dinv_ref[...]


def _tc3_body(p_ref, hp_ref, dinv_ref, b2_ref, g2_ref, be2_ref, wfc_ref,
              bfc_ref, out_ref):
    t = p_ref[0] + p_ref[1] - hp_ref[...]
    agg = t * dinv_ref[...] + b2_ref[...]
    a2 = _bn_relu(agg, g2_ref[...], be2_ref[...])
    out_ref[...] = jnp.dot(a2, wfc_ref[...],
                           preferred_element_type=jnp.float32) + bfc_ref[...]


def kernel(x, edge_index, W1, b1, g1, be1, W2, b2, g2, be2, Wfc, bfc):
    e = edge_index.shape[1]
    egrp = NW * CHUNK * U
    epad = ((e + egrp - 1) // egrp) * egrp
    pad = epad - e
    src = edge_index[0]
    dst = edge_index[1]
    if pad:
        fill = jnp.arange(pad, dtype=jnp.int32)
        src = jnp.concatenate([src, fill % N])
        dst = jnp.concatenate([dst, N + fill % (NPAD - N)])
    n_chunks = epad // CHUNK
    eidx = jnp.stack([src.reshape(n_chunks, CHUNK),
                      dst.reshape(n_chunks, CHUNK)], axis=1)
    x_pad = jnp.pad(x, ((0, NPAD - N), (0, 0)))

    degp = _make_deg_kernel(n_chunks)(eidx)

    hp1, dinv = pl.pallas_call(
        _tc1_body,
        out_shape=[
            jax.ShapeDtypeStruct((NPAD, H1), jnp.float32),
            jax.ShapeDtypeStruct((NPAD, 1), jnp.float32),
        ],
    )(x_pad, W1, degp.T)

    p1 = _make_agg_kernel(n_chunks, H1)(hp1, eidx)

    hp2 = pl.pallas_call(
        _tc2_body,
        out_shape=jax.ShapeDtypeStruct((NPAD, H2), jnp.float32),
    )(p1, hp1, dinv, b1, g1, be1, W2)

    p2 = _make_agg_kernel(n_chunks, H2)(hp2, eidx)

    logits = pl.pallas_call(
        _tc3_body,
        out_shape=jax.ShapeDtypeStruct((NPAD, OUT), jnp.float32),
    )(p2, hp2, dinv, b2, g2, be2, Wfc, bfc)

    return logits[:N]
